# Initial kernel scaffold; baseline (speedup 1.0000x reference)
import jax
import jax.numpy as jnp
from jax.experimental import pallas as pl


def kernel(x, edge_index, edge_attr, batch, W_in, b_in, Wg, bg, We, W_ih, W_hh, b_lstm, W_mu, b_mu, W_lv, b_lv, D1w, D1b, D2w, D2b, D3w, D3b):
    raise NotImplementedError("write your pallas kernel here")


# speedup vs baseline: 2.7509x; 2.7509x over previous
import functools

import jax
import jax.numpy as jnp
from jax import lax
from jax.experimental import pallas as pl
from jax.experimental.pallas import tpu as pltpu
from jax.experimental.pallas import tpu_sc as plsc

N = 10000
E = 160000
D = 256
DE = 16
H = 256
HH = 128
LAT = 128
NG = 64
NL = 4

NC = 2
NS = 16
CH = 128
NCHUNK = 1280
EP = NCHUNK * CH
CPT32 = NCHUNK // (NC * NS)
CPT16 = NCHUNK // NS
NPAD = 10112
ZR = NPAD // NS
PK = 4
PKR = 2560
PZR = PKR // NS
RB = 1000
GRID = N // RB

_f32 = jnp.float32
_mesh = plsc.VectorSubcoreMesh(
    core_axis_name="c", subcore_axis_name="s", num_cores=NC, num_subcores=NS
)


@functools.partial(
    pl.kernel,
    out_type=jax.ShapeDtypeStruct((2 * PKR, HH), _f32),
    mesh=_mesh,
    scratch_types=[
        pltpu.VMEM((CPT32, CH), jnp.int32),
        pltpu.VMEM((CPT32, CH), jnp.int32),
        pltpu.VMEM((CH,), jnp.int32),
        pltpu.VMEM((NPAD // CH, CH), _f32),
        pltpu.VMEM((CH, DE), _f32),
        pltpu.VMEM((CH, HH), _f32),
        pltpu.VMEM_SHARED((PKR, HH), _f32),
    ],
    compiler_params=pltpu.CompilerParams(needs_layout_passes=False),
)
def _sc_cs(src2d, dst2d, ea, dis_p, zpk, out,
           src_t, dst_t, dstbuf, dis_v, ea_t, buf, acc):
    c = lax.axis_index("c")
    s = lax.axis_index("s")
    w = s * NC + c
    pltpu.sync_copy(src2d.at[pl.ds(w * CPT32, CPT32)], src_t)
    pltpu.sync_copy(dst2d.at[pl.ds(w * CPT32, CPT32)], dst_t)
    pltpu.sync_copy(dis_p, dis_v)
    pltpu.sync_copy(zpk, acc.at[pl.ds(s * PZR, PZR)])
    plsc.subcore_barrier()
    lanes = lax.iota(jnp.int32, 16)

    def chunk(k, _):
        pltpu.sync_copy(ea.at[pl.ds((w * CPT32 + k) * CH, CH)], ea_t)
        src_row = src_t.at[k]
        dst_row = dst_t.at[k]
        for g in range(CH // 16):
            rows = lanes + g * 16
            src16 = src_row[pl.ds(g * 16, 16)]
            dst16 = dst_row[pl.ds(g * 16, 16)]
            dstbuf[pl.ds(g * 16, 16)] = lax.shift_right_logical(dst16, 2)
            dis_s = plsc.load_gather(
                dis_v, [lax.shift_right_logical(src16, 7),
                        lax.bitwise_and(src16, 127)])
            dis_d = plsc.load_gather(
                dis_v, [lax.shift_right_logical(dst16, 7),
                        lax.bitwise_and(dst16, 127)])
            nrm = dis_s * dis_d
            pcol = lax.bitwise_and(dst16, PK - 1) * 32
            zero16 = jnp.zeros((16,), _f32)
            for q in range(PK):
                inq = pcol == q * 32
                base = jnp.full((16,), q * 32, jnp.int32)
                for dcol in range(DE):
                    colv = plsc.load_gather(
                        ea_t, [rows, jnp.full((16,), dcol, jnp.int32)])
                    val = jnp.where(inq, colv * nrm, zero16)
                    plsc.store_scatter(buf, [rows, base + dcol], val)
                plsc.store_scatter(buf, [rows, base + DE],
                                   jnp.where(inq, dis_s, zero16))
                for dcol in range(DE + 1, 32):
                    plsc.store_scatter(buf, [rows, base + dcol], zero16)
        pltpu.sync_copy(buf, acc.at[dstbuf], add=True)
        return ()

    lax.fori_loop(0, CPT32, chunk, ())
    plsc.subcore_barrier()
    pltpu.sync_copy(acc.at[pl.ds(s * PZR, PZR)],
                    out.at[pl.ds(c * PKR + s * PZR, PZR)])


@functools.partial(
    pl.kernel,
    out_type=jax.ShapeDtypeStruct((2 * NPAD, HH), _f32),
    mesh=_mesh,
    scratch_types=[
        pltpu.VMEM((CPT16, CH), jnp.int32),
        pltpu.VMEM((CPT16, CH), jnp.int32),
        pltpu.VMEM((CH,), jnp.int32),
        pltpu.VMEM((CH,), jnp.int32),
        pltpu.VMEM((CH, HH), _f32),
        pltpu.VMEM_SHARED((NPAD, HH), _f32),
        pltpu.SemaphoreType.DMA,
    ],
)
def _sc_p(src2d, dst2d, table, z128, out,
          src_t, dst_t, srcbuf, dstbuf, rows_v, acc, sem):
    c = lax.axis_index("c")
    s = lax.axis_index("s")
    cN = c * N
    pltpu.sync_copy(src2d.at[pl.ds(s * CPT16, CPT16)], src_t)
    pltpu.sync_copy(dst2d.at[pl.ds(s * CPT16, CPT16)], dst_t)
    pltpu.sync_copy(z128, acc.at[pl.ds(s * ZR, ZR)])
    plsc.subcore_barrier()

    def chunk(k, _):
        src_row = src_t.at[k]
        dst_row = dst_t.at[k]
        for g in range(CH // 16):
            srcbuf[pl.ds(g * 16, 16)] = src_row[pl.ds(g * 16, 16)] + cN
            dstbuf[pl.ds(g * 16, 16)] = dst_row[pl.ds(g * 16, 16)]
        pltpu.async_copy(table.at[srcbuf], rows_v, sem).wait()
        pltpu.sync_copy(rows_v, acc.at[dstbuf], add=True)
        return ()

    lax.fori_loop(0, CPT16, chunk, ())
    plsc.subcore_barrier()
    pltpu.sync_copy(acc.at[pl.ds(s * ZR, ZR)],
                    out.at[pl.ds(c * NPAD + s * ZR, ZR)])


def _tc1_body(x_ref, dp0_ref, dp1_ref, win_ref, bin_ref,
              dis_ref, ht0_ref, ht1_ref):
    deg = dp0_ref[:, DE:DE + 1] + dp1_ref[:, DE:DE + 1] + 1.0
    dis = 1.0 / jnp.sqrt(deg)
    h0 = jnp.dot(x_ref[...], win_ref[...], preferred_element_type=_f32)
    h0 = jnp.maximum(h0 + bin_ref[...], 0.0)
    ht = dis * h0
    dis_ref[...] = jnp.broadcast_to(dis, (RB, HH))
    ht0_ref[...] = ht[:, :HH]
    ht1_ref[...] = ht[:, HH:]


_tc1 = pl.pallas_call(
    _tc1_body,
    grid=(GRID,),
    in_specs=[
        pl.BlockSpec((RB, D), lambda i: (i, 0)),
        pl.BlockSpec((RB, 2 * DE), lambda i: (i, 0)),
        pl.BlockSpec((RB, 2 * DE), lambda i: (i, 0)),
        pl.BlockSpec((D, H), lambda i: (0, 0)),
        pl.BlockSpec((1, H), lambda i: (0, 0)),
    ],
    out_specs=[
        pl.BlockSpec((RB, HH), lambda i: (i, 0)),
        pl.BlockSpec((RB, HH), lambda i: (i, 0)),
        pl.BlockSpec((RB, HH), lambda i: (i, 0)),
    ],
    out_shape=[
        jax.ShapeDtypeStruct((N, HH), _f32),
        jax.ShapeDtypeStruct((N, HH), _f32),
        jax.ShapeDtypeStruct((N, HH), _f32),
    ],
)


def _tc_layer_body(p0_ref, p1_ref, ht0_ref, ht1_ref, cs0_ref, cs1_ref,
                   dis_ref, wg_ref, bg_ref, we_ref,
                   h_ref, nht0_ref, nht1_ref):
    dis = dis_ref[:, 0:1]
    svec = cs0_ref[:, :DE] + cs1_ref[:, :DE]
    cvec = cs0_ref[:, DE:DE + 1] + cs1_ref[:, DE:DE + 1]
    cc = dis * cvec + dis * dis
    u0 = dis * (p0_ref[...] + ht0_ref[...])
    u1 = dis * (p1_ref[...] + ht1_ref[...])
    wg = wg_ref[...]
    acc = jnp.dot(u0, wg[:HH, :], preferred_element_type=_f32)
    acc = acc + jnp.dot(u1, wg[HH:, :], preferred_element_type=_f32)
    acc = acc + jnp.dot(svec, we_ref[...], preferred_element_type=_f32)
    acc = acc + cc * bg_ref[...]
    h = jnp.maximum(acc, 0.0)
    h_ref[...] = h
    nht0_ref[...] = dis * h[:, :HH]
    nht1_ref[...] = dis * h[:, HH:]


_tc_layer = pl.pallas_call(
    _tc_layer_body,
    grid=(GRID,),
    in_specs=[
        pl.BlockSpec((RB, HH), lambda i: (i, 0)),
        pl.BlockSpec((RB, HH), lambda i: (i, 0)),
        pl.BlockSpec((RB, HH), lambda i: (i, 0)),
        pl.BlockSpec((RB, HH), lambda i: (i, 0)),
        pl.BlockSpec((RB, 2 * DE), lambda i: (i, 0)),
        pl.BlockSpec((RB, 2 * DE), lambda i: (i, 0)),
        pl.BlockSpec((RB, HH), lambda i: (i, 0)),
        pl.BlockSpec((H, H), lambda i: (0, 0)),
        pl.BlockSpec((1, H), lambda i: (0, 0)),
        pl.BlockSpec((DE, H), lambda i: (0, 0)),
    ],
    out_specs=[
        pl.BlockSpec((RB, H), lambda i: (i, 0)),
        pl.BlockSpec((RB, HH), lambda i: (i, 0)),
        pl.BlockSpec((RB, HH), lambda i: (i, 0)),
    ],
    out_shape=[
        jax.ShapeDtypeStruct((N, H), _f32),
        jax.ShapeDtypeStruct((N, HH), _f32),
        jax.ShapeDtypeStruct((N, HH), _f32),
    ],
)


def _tc_fin_body(h_ref, b2d_ref, wih_ref, whh_ref, blstm_ref,
                 wmu_ref, bmu_ref, wlv_ref, blv_ref,
                 d1w_ref, d1b_ref, d2w_ref, d2b_ref, d3w_ref, d3b_ref,
                 eps_ref, z_ref, mu_ref, lv_ref, hd_ref):
    h = h_ref[...]
    bt = b2d_ref[...]
    onehot = (bt == lax.broadcasted_iota(jnp.int32, (1, NG), 1)).astype(_f32)
    wih = wih_ref[...]
    whh = whh_ref[...]
    blstm = blstm_ref[...]
    hs = jnp.zeros((NG, H), _f32)
    cstate = jnp.zeros((NG, H), _f32)
    q_star = jnp.zeros((NG, 2 * H), _f32)
    dn_t = (((0,), (0,)), ((), ()))
    for _ in range(4):
        gates = jnp.dot(q_star, wih, preferred_element_type=_f32)
        gates = gates + jnp.dot(hs, whh, preferred_element_type=_f32) + blstm
        gi = jax.nn.sigmoid(gates[:, :H])
        gf = jax.nn.sigmoid(gates[:, H:2 * H])
        gg = jnp.tanh(gates[:, 2 * H:3 * H])
        go = jax.nn.sigmoid(gates[:, 3 * H:])
        cstate = gf * cstate + gi * gg
        hs = go * jnp.tanh(cstate)
        qb = jnp.dot(onehot, hs, preferred_element_type=_f32)
        e = jnp.sum(h * qb, axis=1, keepdims=True)
        masked = jnp.where(onehot > 0.5, e, -jnp.inf)
        emax = jnp.max(masked, axis=0, keepdims=True)
        emax = jnp.where(jnp.isfinite(emax), emax, 0.0)
        emax_n = jnp.dot(onehot, emax.reshape(NG, 1),
                         preferred_element_type=_f32)
        a = jnp.exp(e - emax_n)
        asum = lax.dot_general(onehot, a, dn_t, preferred_element_type=_f32)
        asum_n = jnp.dot(onehot, asum, preferred_element_type=_f32)
        a = a / (asum_n + 1e-16)
        r = lax.dot_general(onehot, h * a, dn_t, preferred_element_type=_f32)
        q_star = jnp.concatenate([hs, r], axis=1)
    mu = jnp.dot(q_star, wmu_ref[...], preferred_element_type=_f32) + bmu_ref[...]
    lv = jnp.dot(q_star, wlv_ref[...], preferred_element_type=_f32) + blv_ref[...]
    z = mu + eps_ref[...] * jnp.exp(0.5 * lv)
    d = jnp.maximum(jnp.dot(z, d1w_ref[...], preferred_element_type=_f32)
                    + d1b_ref[...], 0.0)
    d = jnp.maximum(jnp.dot(d, d2w_ref[...], preferred_element_type=_f32)
                    + d2b_ref[...], 0.0)
    hd = jnp.dot(d, d3w_ref[...], preferred_element_type=_f32) + d3b_ref[...]
    z_ref[...] = z
    mu_ref[...] = mu
    lv_ref[...] = lv
    hd_ref[...] = hd


_tc_fin = pl.pallas_call(
    _tc_fin_body,
    out_shape=[
        jax.ShapeDtypeStruct((NG, LAT), _f32),
        jax.ShapeDtypeStruct((NG, LAT), _f32),
        jax.ShapeDtypeStruct((NG, LAT), _f32),
        jax.ShapeDtypeStruct((NG, 2 * H), _f32),
    ],
)


def _depack(o):
    o0 = o[:PKR].reshape(PKR * PK, 32)[:N]
    o1 = o[PKR:].reshape(PKR * PK, 32)[:N]
    return o0, o1


def kernel(x, edge_index, edge_attr, batch, W_in, b_in, Wg, bg, We,
           W_ih, W_hh, b_lstm, W_mu, b_mu, W_lv, b_lv,
           D1w, D1b, D2w, D2b, D3w, D3b):
    src = edge_index[0]
    dst = edge_index[1]
    pad = EP - E
    src_p = jnp.concatenate(
        [src, jnp.zeros((pad,), jnp.int32)]).reshape(NCHUNK, CH)
    dst_p = jnp.concatenate(
        [dst, jnp.full((pad,), N, jnp.int32)]).reshape(NCHUNK, CH)
    ea_p = jnp.concatenate([edge_attr, jnp.zeros((pad, DE), _f32)], axis=0)
    zpk = jnp.zeros((PZR, HH), _f32)
    z128 = jnp.zeros((ZR, HH), _f32)
    ones_dis = jnp.ones((NPAD // CH, CH), _f32)

    dp0, dp1 = _depack(_sc_cs(src_p, dst_p, ea_p, ones_dis, zpk))
    dis_b, ht0, ht1 = _tc1(x, dp0, dp1, W_in, b_in.reshape(1, H))
    dis_pad = jnp.concatenate(
        [dis_b[:, 0], jnp.ones((NPAD - N,), _f32)]).reshape(NPAD // CH, CH)
    cs0, cs1 = _depack(_sc_cs(src_p, dst_p, ea_p, dis_pad, zpk))

    h = None
    for l in range(NL):
        table = jnp.concatenate([ht0, ht1], axis=0)
        pout = _sc_p(src_p, dst_p, table, z128)
        p0, p1 = pout[:N], pout[NPAD:NPAD + N]
        h, ht0, ht1 = _tc_layer(p0, p1, ht0, ht1, cs0, cs1, dis_b,
                                Wg[l], bg[l].reshape(1, H), We[l])

    eps = jax.random.normal(jax.random.key(42), (NG, LAT), dtype=_f32)
    z, mu, lv, hd = _tc_fin(
        h, batch.reshape(N, 1), W_ih, W_hh, b_lstm.reshape(1, 4 * H),
        W_mu, b_mu.reshape(1, LAT), W_lv, b_lv.reshape(1, LAT),
        D1w, D1b.reshape(1, H), D2w, D2b.reshape(1, H),
        D3w, D3b.reshape(1, 2 * H), eps)
    return (z, mu, lv, hd, h)


# speedup vs baseline: 3.6588x; 1.3300x over previous
import functools

import jax
import jax.numpy as jnp
from jax import lax
from jax.experimental import pallas as pl
from jax.experimental.pallas import tpu as pltpu
from jax.experimental.pallas import tpu_sc as plsc

N = 10000
E = 160000
D = 256
DE = 16
H = 256
HH = 128
LAT = 128
NG = 64
NL = 4

NC = 2
NS = 16
CH = 128
NCHUNK = 1280
EP = NCHUNK * CH
CPT32 = NCHUNK // (NC * NS)
CPT16 = NCHUNK // NS
NPAD = 10112
ZR = NPAD // NS
PK = 4
PKR = 2560
PZR = PKR // NS
RB = 1000
GRID = N // RB

_f32 = jnp.float32
_mesh = plsc.VectorSubcoreMesh(
    core_axis_name="c", subcore_axis_name="s", num_cores=NC, num_subcores=NS
)


@functools.partial(
    pl.kernel,
    out_type=jax.ShapeDtypeStruct((2 * PKR, HH), _f32),
    mesh=_mesh,
    scratch_types=[
        pltpu.VMEM((CPT32, CH), jnp.int32),
        pltpu.VMEM((CPT32, CH), jnp.int32),
        pltpu.VMEM((CH,), jnp.int32),
        pltpu.VMEM((CH,), jnp.int32),
        pltpu.VMEM((NPAD // CH, CH), _f32),
        pltpu.VMEM((CH, DE), _f32),
        pltpu.VMEM((CH, DE), _f32),
        pltpu.VMEM((CH, HH), _f32),
        pltpu.VMEM((CH, HH), _f32),
        pltpu.VMEM_SHARED((PKR, HH), _f32),
        pltpu.SemaphoreType.DMA,
        pltpu.SemaphoreType.DMA,
        pltpu.SemaphoreType.DMA,
        pltpu.SemaphoreType.DMA,
    ],
    compiler_params=pltpu.CompilerParams(needs_layout_passes=False),
)
def _sc_cs(src2d, dst2d, eat, dis_p, zpk, out,
           src_t, dst_t, dstbuf0, dstbuf1, dis_v, ea0, ea1, buf0, buf1,
           acc, sg0, sg1, ss0, ss1):
    c = lax.axis_index("c")
    s = lax.axis_index("s")
    w = s * NC + c
    pltpu.sync_copy(src2d.at[pl.ds(w * CPT32, CPT32)], src_t)
    pltpu.sync_copy(dst2d.at[pl.ds(w * CPT32, CPT32)], dst_t)
    pltpu.sync_copy(dis_p, dis_v)
    pltpu.sync_copy(zpk, acc.at[pl.ds(s * PZR, PZR)])
    zero16 = jnp.zeros((16,), _f32)

    def zb(j, _):
        for hcol in range(HH // 16):
            buf0[j, pl.ds(hcol * 16, 16)] = zero16
            buf1[j, pl.ds(hcol * 16, 16)] = zero16
        return ()

    lax.fori_loop(0, CH, zb, ())
    plsc.subcore_barrier()
    lanes = lax.iota(jnp.int32, 16)

    def fill(k, buf, dstbuf, ea_t):
        src_row = src_t.at[k]
        dst_row = dst_t.at[k]
        for g in range(CH // 16):
            rows = lanes + g * 16
            src16 = src_row[pl.ds(g * 16, 16)]
            dst16 = dst_row[pl.ds(g * 16, 16)]
            dstbuf[pl.ds(g * 16, 16)] = lax.shift_right_logical(dst16, 2)
            dis_s = plsc.load_gather(
                dis_v, [lax.shift_right_logical(src16, 7),
                        lax.bitwise_and(src16, 127)])
            dis_d = plsc.load_gather(
                dis_v, [lax.shift_right_logical(dst16, 7),
                        lax.bitwise_and(dst16, 127)])
            nrm = dis_s * dis_d
            pcol = lax.bitwise_and(dst16, PK - 1) * 32
            op1 = lax.bitwise_and(pcol + 32, 127)
            op2 = lax.bitwise_and(pcol + 64, 127)
            op3 = lax.bitwise_and(pcol + 96, 127)
            for dcol in range(DE):
                colv = plsc.load_gather(
                    ea_t, [rows, jnp.full((16,), dcol, jnp.int32)])
                plsc.store_scatter(buf, [rows, pcol + dcol], colv * nrm)
                plsc.store_scatter(buf, [rows, op1 + dcol], zero16)
                plsc.store_scatter(buf, [rows, op2 + dcol], zero16)
                plsc.store_scatter(buf, [rows, op3 + dcol], zero16)
            plsc.store_scatter(buf, [rows, pcol + DE], dis_s)
            plsc.store_scatter(buf, [rows, op1 + DE], zero16)
            plsc.store_scatter(buf, [rows, op2 + DE], zero16)
            plsc.store_scatter(buf, [rows, op3 + DE], zero16)

    def body(k2, _):
        k0 = 2 * k2
        k1 = 2 * k2 + 1
        g0 = pltpu.async_copy(
            eat.at[pl.ds((w * CPT32 + k0) * CH, CH)], ea0, sg0)
        g1 = pltpu.async_copy(
            eat.at[pl.ds((w * CPT32 + k1) * CH, CH)], ea1, sg1)
        g0.wait()
        fill(k0, buf0, dstbuf0, ea0)
        w0 = pltpu.async_copy(buf0, acc.at[dstbuf0], ss0, add=True)
        g1.wait()
        fill(k1, buf1, dstbuf1, ea1)
        w1 = pltpu.async_copy(buf1, acc.at[dstbuf1], ss1, add=True)
        w0.wait()
        w1.wait()
        return ()

    lax.fori_loop(0, CPT32 // 2, body, ())
    plsc.subcore_barrier()
    pltpu.sync_copy(acc.at[pl.ds(s * PZR, PZR)],
                    out.at[pl.ds(c * PKR + s * PZR, PZR)])


NSLOT = 2

@functools.partial(
    pl.kernel,
    out_type=jax.ShapeDtypeStruct((2 * NPAD, HH), _f32),
    mesh=_mesh,
    scratch_types=(
        [pltpu.VMEM((1, CH), jnp.int32)] * NSLOT
        + [pltpu.VMEM((CH,), jnp.int32)] * (2 * NSLOT)
        + [pltpu.VMEM((CH, HH), _f32)] * NSLOT
        + [pltpu.VMEM_SHARED((NPAD, HH), _f32)]
        + [pltpu.SemaphoreType.DMA] * (2 * NSLOT)
    ),
)
def _sc_p(sd3d, table, z128, out, *rest):
    sdst = rest[0:NSLOT]
    srcbufs = rest[NSLOT:2 * NSLOT]
    dstbufs = rest[2 * NSLOT:3 * NSLOT]
    rows = rest[3 * NSLOT:4 * NSLOT]
    acc = rest[4 * NSLOT]
    isems = rest[4 * NSLOT + 1:5 * NSLOT + 1]
    gsems = rest[5 * NSLOT + 1:]
    c = lax.axis_index("c")
    s = lax.axis_index("s")
    cN = c * N
    pltpu.sync_copy(z128, acc.at[pl.ds(s * ZR, ZR)])
    plsc.subcore_barrier()

    def body(k, _):
        base = s * CPT16 + k * NSLOT
        ih = [pltpu.async_copy(sd3d.at[base + j], sdst[j], isems[j])
              for j in range(NSLOT)]
        gh = []
        for j in range(NSLOT):
            ih[j].wait()
            sd_row = sdst[j].at[0]
            for g in range(CH // 16):
                v = sd_row[pl.ds(g * 16, 16)]
                srcbufs[j][pl.ds(g * 16, 16)] = (
                    lax.bitwise_and(v, 16383) + cN)
                dstbufs[j][pl.ds(g * 16, 16)] = (
                    lax.shift_right_logical(v, 14))
            gh.append(pltpu.async_copy(
                table.at[srcbufs[j]], rows[j], gsems[j]))
        for j in range(NSLOT):
            gh[j].wait()
            pltpu.sync_copy(rows[j], acc.at[dstbufs[j]], add=True)
        return ()

    lax.fori_loop(0, CPT16 // NSLOT, body, ())
    plsc.subcore_barrier()
    pltpu.sync_copy(acc.at[pl.ds(s * ZR, ZR)],
                    out.at[pl.ds(c * NPAD + s * ZR, ZR)])


def _tc1_body(x_ref, dp0_ref, dp1_ref, win_ref, bin_ref,
              dis_ref, ht0_ref, ht1_ref):
    deg = dp0_ref[:, DE:DE + 1] + dp1_ref[:, DE:DE + 1] + 1.0
    dis = 1.0 / jnp.sqrt(deg)
    h0 = jnp.dot(x_ref[...], win_ref[...], preferred_element_type=_f32)
    h0 = jnp.maximum(h0 + bin_ref[...], 0.0)
    ht = dis * h0
    dis_ref[...] = jnp.broadcast_to(dis, (RB, HH))
    ht0_ref[...] = ht[:, :HH]
    ht1_ref[...] = ht[:, HH:]


_tc1 = pl.pallas_call(
    _tc1_body,
    grid=(GRID,),
    in_specs=[
        pl.BlockSpec((RB, D), lambda i: (i, 0)),
        pl.BlockSpec((RB, 2 * DE), lambda i: (i, 0)),
        pl.BlockSpec((RB, 2 * DE), lambda i: (i, 0)),
        pl.BlockSpec((D, H), lambda i: (0, 0)),
        pl.BlockSpec((1, H), lambda i: (0, 0)),
    ],
    out_specs=[
        pl.BlockSpec((RB, HH), lambda i: (i, 0)),
        pl.BlockSpec((RB, HH), lambda i: (i, 0)),
        pl.BlockSpec((RB, HH), lambda i: (i, 0)),
    ],
    out_shape=[
        jax.ShapeDtypeStruct((N, HH), _f32),
        jax.ShapeDtypeStruct((N, HH), _f32),
        jax.ShapeDtypeStruct((N, HH), _f32),
    ],
)


def _tc_layer_body(p0_ref, p1_ref, ht0_ref, ht1_ref, cs0_ref, cs1_ref,
                   dis_ref, wg_ref, bg_ref, we_ref,
                   h_ref, nht0_ref, nht1_ref):
    dis = dis_ref[:, 0:1]
    svec = cs0_ref[:, :DE] + cs1_ref[:, :DE]
    cvec = cs0_ref[:, DE:DE + 1] + cs1_ref[:, DE:DE + 1]
    cc = dis * cvec + dis * dis
    u0 = dis * (p0_ref[...] + ht0_ref[...])
    u1 = dis * (p1_ref[...] + ht1_ref[...])
    wg = wg_ref[...]
    acc = jnp.dot(u0, wg[:HH, :], preferred_element_type=_f32)
    acc = acc + jnp.dot(u1, wg[HH:, :], preferred_element_type=_f32)
    acc = acc + jnp.dot(svec, we_ref[...], preferred_element_type=_f32)
    acc = acc + cc * bg_ref[...]
    h = jnp.maximum(acc, 0.0)
    h_ref[...] = h
    nht0_ref[...] = dis * h[:, :HH]
    nht1_ref[...] = dis * h[:, HH:]


_tc_layer = pl.pallas_call(
    _tc_layer_body,
    grid=(GRID,),
    in_specs=[
        pl.BlockSpec((RB, HH), lambda i: (i, 0)),
        pl.BlockSpec((RB, HH), lambda i: (i, 0)),
        pl.BlockSpec((RB, HH), lambda i: (i, 0)),
        pl.BlockSpec((RB, HH), lambda i: (i, 0)),
        pl.BlockSpec((RB, 2 * DE), lambda i: (i, 0)),
        pl.BlockSpec((RB, 2 * DE), lambda i: (i, 0)),
        pl.BlockSpec((RB, HH), lambda i: (i, 0)),
        pl.BlockSpec((H, H), lambda i: (0, 0)),
        pl.BlockSpec((1, H), lambda i: (0, 0)),
        pl.BlockSpec((DE, H), lambda i: (0, 0)),
    ],
    out_specs=[
        pl.BlockSpec((RB, H), lambda i: (i, 0)),
        pl.BlockSpec((RB, HH), lambda i: (i, 0)),
        pl.BlockSpec((RB, HH), lambda i: (i, 0)),
    ],
    out_shape=[
        jax.ShapeDtypeStruct((N, H), _f32),
        jax.ShapeDtypeStruct((N, HH), _f32),
        jax.ShapeDtypeStruct((N, HH), _f32),
    ],
)


def _tc_fin_body(h_ref, b2d_ref, wih_ref, whh_ref, blstm_ref,
                 wmu_ref, bmu_ref, wlv_ref, blv_ref,
                 d1w_ref, d1b_ref, d2w_ref, d2b_ref, d3w_ref, d3b_ref,
                 eps_ref, z_ref, mu_ref, lv_ref, hd_ref):
    h = h_ref[...]
    bt = b2d_ref[...]
    onehot = (bt == lax.broadcasted_iota(jnp.int32, (1, NG), 1)).astype(_f32)
    wih = wih_ref[...]
    whh = whh_ref[...]
    blstm = blstm_ref[...]
    hs = jnp.zeros((NG, H), _f32)
    cstate = jnp.zeros((NG, H), _f32)
    q_star = jnp.zeros((NG, 2 * H), _f32)
    dn_t = (((0,), (0,)), ((), ()))
    for _ in range(4):
        gates = jnp.dot(q_star, wih, preferred_element_type=_f32)
        gates = gates + jnp.dot(hs, whh, preferred_element_type=_f32) + blstm
        gi = jax.nn.sigmoid(gates[:, :H])
        gf = jax.nn.sigmoid(gates[:, H:2 * H])
        gg = jnp.tanh(gates[:, 2 * H:3 * H])
        go = jax.nn.sigmoid(gates[:, 3 * H:])
        cstate = gf * cstate + gi * gg
        hs = go * jnp.tanh(cstate)
        qb = jnp.dot(onehot, hs, preferred_element_type=_f32)
        e = jnp.sum(h * qb, axis=1, keepdims=True)
        masked = jnp.where(onehot > 0.5, e, -jnp.inf)
        emax = jnp.max(masked, axis=0, keepdims=True)
        emax = jnp.where(jnp.isfinite(emax), emax, 0.0)
        emax_n = jnp.dot(onehot, emax.reshape(NG, 1),
                         preferred_element_type=_f32)
        a = jnp.exp(e - emax_n)
        asum = lax.dot_general(onehot, a, dn_t, preferred_element_type=_f32)
        asum_n = jnp.dot(onehot, asum, preferred_element_type=_f32)
        a = a / (asum_n + 1e-16)
        r = lax.dot_general(onehot, h * a, dn_t, preferred_element_type=_f32)
        q_star = jnp.concatenate([hs, r], axis=1)
    mu = jnp.dot(q_star, wmu_ref[...], preferred_element_type=_f32) + bmu_ref[...]
    lv = jnp.dot(q_star, wlv_ref[...], preferred_element_type=_f32) + blv_ref[...]
    z = mu + eps_ref[...] * jnp.exp(0.5 * lv)
    d = jnp.maximum(jnp.dot(z, d1w_ref[...], preferred_element_type=_f32)
                    + d1b_ref[...], 0.0)
    d = jnp.maximum(jnp.dot(d, d2w_ref[...], preferred_element_type=_f32)
                    + d2b_ref[...], 0.0)
    hd = jnp.dot(d, d3w_ref[...], preferred_element_type=_f32) + d3b_ref[...]
    z_ref[...] = z
    mu_ref[...] = mu
    lv_ref[...] = lv
    hd_ref[...] = hd


_tc_fin = pl.pallas_call(
    _tc_fin_body,
    out_shape=[
        jax.ShapeDtypeStruct((NG, LAT), _f32),
        jax.ShapeDtypeStruct((NG, LAT), _f32),
        jax.ShapeDtypeStruct((NG, LAT), _f32),
        jax.ShapeDtypeStruct((NG, 2 * H), _f32),
    ],
)


def _depack(o):
    o0 = o[:PKR].reshape(PKR * PK, 32)[:N]
    o1 = o[PKR:].reshape(PKR * PK, 32)[:N]
    return o0, o1


def kernel(x, edge_index, edge_attr, batch, W_in, b_in, Wg, bg, We,
           W_ih, W_hh, b_lstm, W_mu, b_mu, W_lv, b_lv,
           D1w, D1b, D2w, D2b, D3w, D3b):
    src = edge_index[0]
    dst = edge_index[1]
    pad = EP - E
    src_p = jnp.concatenate(
        [src, jnp.zeros((pad,), jnp.int32)]).reshape(NCHUNK, CH)
    dst_p = jnp.concatenate(
        [dst, jnp.full((pad,), N, jnp.int32)]).reshape(NCHUNK, CH)
    ea_t = jnp.concatenate(
        [edge_attr, jnp.zeros((pad, DE), _f32)], axis=0)
    sd_p = (src_p + dst_p * 16384).reshape(NCHUNK, 1, CH)
    zpk = jnp.zeros((PZR, HH), _f32)
    z128 = jnp.zeros((ZR, HH), _f32)
    ones_dis = jnp.ones((NPAD // CH, CH), _f32)

    dp0, dp1 = _depack(_sc_cs(src_p, dst_p, ea_t, ones_dis, zpk))
    dis_b, ht0, ht1 = _tc1(x, dp0, dp1, W_in, b_in.reshape(1, H))
    dis_pad = jnp.concatenate(
        [dis_b[:, 0], jnp.ones((NPAD - N,), _f32)]).reshape(NPAD // CH, CH)
    cs0, cs1 = _depack(_sc_cs(src_p, dst_p, ea_t, dis_pad, zpk))

    h = None
    for l in range(NL):
        table = jnp.concatenate([ht0, ht1], axis=0)
        pout = _sc_p(sd_p, table, z128)
        p0, p1 = pout[:N], pout[NPAD:NPAD + N]
        h, ht0, ht1 = _tc_layer(p0, p1, ht0, ht1, cs0, cs1, dis_b,
                                Wg[l], bg[l].reshape(1, H), We[l])

    eps = jax.random.normal(jax.random.key(42), (NG, LAT), dtype=_f32)
    z, mu, lv, hd = _tc_fin(
        h, batch.reshape(N, 1), W_ih, W_hh, b_lstm.reshape(1, 4 * H),
        W_mu, b_mu.reshape(1, LAT), W_lv, b_lv.reshape(1, LAT),
        D1w, D1b.reshape(1, H), D2w, D2b.reshape(1, H),
        D3w, D3b.reshape(1, 2 * H), eps)
    return (z, mu, lv, hd, h)


# speedup vs baseline: 3.6701x; 1.0031x over previous
import functools

import jax
import jax.numpy as jnp
from jax import lax
from jax.experimental import pallas as pl
from jax.experimental.pallas import tpu as pltpu
from jax.experimental.pallas import tpu_sc as plsc

N = 10000
E = 160000
D = 256
DE = 16
H = 256
HH = 128
LAT = 128
NG = 64
NL = 4

NC = 2
NS = 16
CH = 128
NCHUNK = 1280
EP = NCHUNK * CH
CPT32 = NCHUNK // (NC * NS)
CPT16 = NCHUNK // NS
NPAD = 10112
ZR = NPAD // NS
PK = 4
PKR = 2560
PZR = PKR // NS
RB = 1000
GRID = N // RB

_f32 = jnp.float32
_mesh = plsc.VectorSubcoreMesh(
    core_axis_name="c", subcore_axis_name="s", num_cores=NC, num_subcores=NS
)


@functools.partial(
    pl.kernel,
    out_type=jax.ShapeDtypeStruct((2 * PKR, HH), _f32),
    mesh=_mesh,
    scratch_types=[
        pltpu.VMEM((CPT32, CH), jnp.int32),
        pltpu.VMEM((CPT32, CH), jnp.int32),
        pltpu.VMEM((CH,), jnp.int32),
        pltpu.VMEM((CH,), jnp.int32),
        pltpu.VMEM((NPAD // CH, CH), _f32),
        pltpu.VMEM((CH, DE), _f32),
        pltpu.VMEM((CH, DE), _f32),
        pltpu.VMEM((CH, HH), _f32),
        pltpu.VMEM((CH, HH), _f32),
        pltpu.VMEM_SHARED((PKR, HH), _f32),
        pltpu.SemaphoreType.DMA,
        pltpu.SemaphoreType.DMA,
        pltpu.SemaphoreType.DMA,
        pltpu.SemaphoreType.DMA,
    ],
    compiler_params=pltpu.CompilerParams(needs_layout_passes=False),
)
def _sc_cs(src2d, dst2d, eat, dis_p, zpk, out,
           src_t, dst_t, dstbuf0, dstbuf1, dis_v, ea0, ea1, buf0, buf1,
           acc, sg0, sg1, ss0, ss1):
    c = lax.axis_index("c")
    s = lax.axis_index("s")
    w = s * NC + c
    pltpu.sync_copy(src2d.at[pl.ds(w * CPT32, CPT32)], src_t)
    pltpu.sync_copy(dst2d.at[pl.ds(w * CPT32, CPT32)], dst_t)
    pltpu.sync_copy(dis_p, dis_v)
    pltpu.sync_copy(zpk, acc.at[pl.ds(s * PZR, PZR)])
    zero16 = jnp.zeros((16,), _f32)

    def zb(j, _):
        for hcol in range(HH // 16):
            buf0[j, pl.ds(hcol * 16, 16)] = zero16
            buf1[j, pl.ds(hcol * 16, 16)] = zero16
        return ()

    lax.fori_loop(0, CH, zb, ())
    plsc.subcore_barrier()
    lanes = lax.iota(jnp.int32, 16)

    def fill(k, buf, dstbuf, ea_t):
        src_row = src_t.at[k]
        dst_row = dst_t.at[k]
        for g in range(CH // 16):
            rows = lanes + g * 16
            src16 = src_row[pl.ds(g * 16, 16)]
            dst16 = dst_row[pl.ds(g * 16, 16)]
            dstbuf[pl.ds(g * 16, 16)] = lax.shift_right_logical(dst16, 2)
            dis_s = plsc.load_gather(
                dis_v, [lax.shift_right_logical(src16, 7),
                        lax.bitwise_and(src16, 127)])
            dis_d = plsc.load_gather(
                dis_v, [lax.shift_right_logical(dst16, 7),
                        lax.bitwise_and(dst16, 127)])
            nrm = dis_s * dis_d
            pcol = lax.bitwise_and(dst16, PK - 1) * 32
            op1 = lax.bitwise_and(pcol + 32, 127)
            op2 = lax.bitwise_and(pcol + 64, 127)
            op3 = lax.bitwise_and(pcol + 96, 127)
            for dcol in range(DE):
                colv = plsc.load_gather(
                    ea_t, [rows, jnp.full((16,), dcol, jnp.int32)])
                plsc.store_scatter(buf, [rows, pcol + dcol], colv * nrm)
                plsc.store_scatter(buf, [rows, op1 + dcol], zero16)
                plsc.store_scatter(buf, [rows, op2 + dcol], zero16)
                plsc.store_scatter(buf, [rows, op3 + dcol], zero16)
            plsc.store_scatter(buf, [rows, pcol + DE], dis_s)
            plsc.store_scatter(buf, [rows, op1 + DE], zero16)
            plsc.store_scatter(buf, [rows, op2 + DE], zero16)
            plsc.store_scatter(buf, [rows, op3 + DE], zero16)

    def body(k2, _):
        k0 = 2 * k2
        k1 = 2 * k2 + 1
        g0 = pltpu.async_copy(
            eat.at[pl.ds((w * CPT32 + k0) * CH, CH)], ea0, sg0)
        g1 = pltpu.async_copy(
            eat.at[pl.ds((w * CPT32 + k1) * CH, CH)], ea1, sg1)
        g0.wait()
        fill(k0, buf0, dstbuf0, ea0)
        w0 = pltpu.async_copy(buf0, acc.at[dstbuf0], ss0, add=True)
        g1.wait()
        fill(k1, buf1, dstbuf1, ea1)
        w1 = pltpu.async_copy(buf1, acc.at[dstbuf1], ss1, add=True)
        w0.wait()
        w1.wait()
        return ()

    lax.fori_loop(0, CPT32 // 2, body, ())
    plsc.subcore_barrier()
    pltpu.sync_copy(acc.at[pl.ds(s * PZR, PZR)],
                    out.at[pl.ds(c * PKR + s * PZR, PZR)])


NSLOT = 2

@functools.partial(
    pl.kernel,
    out_type=jax.ShapeDtypeStruct((2 * NPAD, HH), _f32),
    mesh=_mesh,
    scratch_types=(
        [pltpu.VMEM((1, CH), jnp.int32)] * NSLOT
        + [pltpu.VMEM((CH,), jnp.int32)] * (2 * NSLOT)
        + [pltpu.VMEM((CH, HH), _f32)] * NSLOT
        + [pltpu.VMEM_SHARED((NPAD, HH), _f32)]
        + [pltpu.SemaphoreType.DMA] * (3 * NSLOT)
    ),
)
def _sc_p(sd3d, table, z128, out, *rest):
    sdst = rest[0:NSLOT]
    srcbufs = rest[NSLOT:2 * NSLOT]
    dstbufs = rest[2 * NSLOT:3 * NSLOT]
    rows = rest[3 * NSLOT:4 * NSLOT]
    acc = rest[4 * NSLOT]
    isems = rest[4 * NSLOT + 1:5 * NSLOT + 1]
    gsems = rest[5 * NSLOT + 1:6 * NSLOT + 1]
    ssems = rest[6 * NSLOT + 1:]
    c = lax.axis_index("c")
    s = lax.axis_index("s")
    cN = c * N
    pltpu.sync_copy(z128, acc.at[pl.ds(s * ZR, ZR)])
    plsc.subcore_barrier()

    def body(k, _):
        base = s * CPT16 + k * NSLOT
        ih = [pltpu.async_copy(sd3d.at[base + j], sdst[j], isems[j])
              for j in range(NSLOT)]
        gh = []
        for j in range(NSLOT):
            ih[j].wait()
            sd_row = sdst[j].at[0]
            for g in range(CH // 16):
                v = sd_row[pl.ds(g * 16, 16)]
                srcbufs[j][pl.ds(g * 16, 16)] = (
                    lax.bitwise_and(v, 16383) + cN)
                dstbufs[j][pl.ds(g * 16, 16)] = (
                    lax.shift_right_logical(v, 14))
            gh.append(pltpu.async_copy(
                table.at[srcbufs[j]], rows[j], gsems[j]))
        sh = []
        for j in range(NSLOT):
            gh[j].wait()
            sh.append(pltpu.async_copy(
                rows[j], acc.at[dstbufs[j]], ssems[j], add=True))
        for j in range(NSLOT):
            sh[j].wait()
        return ()

    lax.fori_loop(0, CPT16 // NSLOT, body, ())
    plsc.subcore_barrier()
    pltpu.sync_copy(acc.at[pl.ds(s * ZR, ZR)],
                    out.at[pl.ds(c * NPAD + s * ZR, ZR)])


def _tc1_body(x_ref, dp0_ref, dp1_ref, win_ref, bin_ref,
              dis_ref, ht0_ref, ht1_ref):
    deg = dp0_ref[:, DE:DE + 1] + dp1_ref[:, DE:DE + 1] + 1.0
    dis = 1.0 / jnp.sqrt(deg)
    h0 = jnp.dot(x_ref[...], win_ref[...], preferred_element_type=_f32)
    h0 = jnp.maximum(h0 + bin_ref[...], 0.0)
    ht = dis * h0
    dis_ref[...] = jnp.broadcast_to(dis, (RB, HH))
    ht0_ref[...] = ht[:, :HH]
    ht1_ref[...] = ht[:, HH:]


_tc1 = pl.pallas_call(
    _tc1_body,
    grid=(GRID,),
    in_specs=[
        pl.BlockSpec((RB, D), lambda i: (i, 0)),
        pl.BlockSpec((RB, 2 * DE), lambda i: (i, 0)),
        pl.BlockSpec((RB, 2 * DE), lambda i: (i, 0)),
        pl.BlockSpec((D, H), lambda i: (0, 0)),
        pl.BlockSpec((1, H), lambda i: (0, 0)),
    ],
    out_specs=[
        pl.BlockSpec((RB, HH), lambda i: (i, 0)),
        pl.BlockSpec((RB, HH), lambda i: (i, 0)),
        pl.BlockSpec((RB, HH), lambda i: (i, 0)),
    ],
    out_shape=[
        jax.ShapeDtypeStruct((N, HH), _f32),
        jax.ShapeDtypeStruct((N, HH), _f32),
        jax.ShapeDtypeStruct((N, HH), _f32),
    ],
)


def _tc_layer_body(p0_ref, p1_ref, ht0_ref, ht1_ref, cs0_ref, cs1_ref,
                   dis_ref, wg_ref, bg_ref, we_ref,
                   h_ref, nht0_ref, nht1_ref):
    dis = dis_ref[:, 0:1]
    svec = cs0_ref[:, :DE] + cs1_ref[:, :DE]
    cvec = cs0_ref[:, DE:DE + 1] + cs1_ref[:, DE:DE + 1]
    cc = dis * cvec + dis * dis
    u0 = dis * (p0_ref[...] + ht0_ref[...])
    u1 = dis * (p1_ref[...] + ht1_ref[...])
    wg = wg_ref[...]
    acc = jnp.dot(u0, wg[:HH, :], preferred_element_type=_f32)
    acc = acc + jnp.dot(u1, wg[HH:, :], preferred_element_type=_f32)
    acc = acc + jnp.dot(svec, we_ref[...], preferred_element_type=_f32)
    acc = acc + cc * bg_ref[...]
    h = jnp.maximum(acc, 0.0)
    h_ref[...] = h
    nht0_ref[...] = dis * h[:, :HH]
    nht1_ref[...] = dis * h[:, HH:]


_tc_layer = pl.pallas_call(
    _tc_layer_body,
    grid=(GRID,),
    in_specs=[
        pl.BlockSpec((RB, HH), lambda i: (i, 0)),
        pl.BlockSpec((RB, HH), lambda i: (i, 0)),
        pl.BlockSpec((RB, HH), lambda i: (i, 0)),
        pl.BlockSpec((RB, HH), lambda i: (i, 0)),
        pl.BlockSpec((RB, 2 * DE), lambda i: (i, 0)),
        pl.BlockSpec((RB, 2 * DE), lambda i: (i, 0)),
        pl.BlockSpec((RB, HH), lambda i: (i, 0)),
        pl.BlockSpec((H, H), lambda i: (0, 0)),
        pl.BlockSpec((1, H), lambda i: (0, 0)),
        pl.BlockSpec((DE, H), lambda i: (0, 0)),
    ],
    out_specs=[
        pl.BlockSpec((RB, H), lambda i: (i, 0)),
        pl.BlockSpec((RB, HH), lambda i: (i, 0)),
        pl.BlockSpec((RB, HH), lambda i: (i, 0)),
    ],
    out_shape=[
        jax.ShapeDtypeStruct((N, H), _f32),
        jax.ShapeDtypeStruct((N, HH), _f32),
        jax.ShapeDtypeStruct((N, HH), _f32),
    ],
)


def _tc_fin_body(h_ref, b2d_ref, wih_ref, whh_ref, blstm_ref,
                 wmu_ref, bmu_ref, wlv_ref, blv_ref,
                 d1w_ref, d1b_ref, d2w_ref, d2b_ref, d3w_ref, d3b_ref,
                 eps_ref, z_ref, mu_ref, lv_ref, hd_ref):
    h = h_ref[...]
    bt = b2d_ref[...]
    onehot = (bt == lax.broadcasted_iota(jnp.int32, (1, NG), 1)).astype(_f32)
    wih = wih_ref[...]
    whh = whh_ref[...]
    blstm = blstm_ref[...]
    hs = jnp.zeros((NG, H), _f32)
    cstate = jnp.zeros((NG, H), _f32)
    q_star = jnp.zeros((NG, 2 * H), _f32)
    dn_t = (((0,), (0,)), ((), ()))
    for _ in range(4):
        gates = jnp.dot(q_star, wih, preferred_element_type=_f32)
        gates = gates + jnp.dot(hs, whh, preferred_element_type=_f32) + blstm
        gi = jax.nn.sigmoid(gates[:, :H])
        gf = jax.nn.sigmoid(gates[:, H:2 * H])
        gg = jnp.tanh(gates[:, 2 * H:3 * H])
        go = jax.nn.sigmoid(gates[:, 3 * H:])
        cstate = gf * cstate + gi * gg
        hs = go * jnp.tanh(cstate)
        qb = jnp.dot(onehot, hs, preferred_element_type=_f32)
        e = jnp.sum(h * qb, axis=1, keepdims=True)
        masked = jnp.where(onehot > 0.5, e, -jnp.inf)
        emax = jnp.max(masked, axis=0, keepdims=True)
        emax = jnp.where(jnp.isfinite(emax), emax, 0.0)
        emax_n = jnp.dot(onehot, emax.reshape(NG, 1),
                         preferred_element_type=_f32)
        a = jnp.exp(e - emax_n)
        asum = lax.dot_general(onehot, a, dn_t, preferred_element_type=_f32)
        asum_n = jnp.dot(onehot, asum, preferred_element_type=_f32)
        a = a / (asum_n + 1e-16)
        r = lax.dot_general(onehot, h * a, dn_t, preferred_element_type=_f32)
        q_star = jnp.concatenate([hs, r], axis=1)
    mu = jnp.dot(q_star, wmu_ref[...], preferred_element_type=_f32) + bmu_ref[...]
    lv = jnp.dot(q_star, wlv_ref[...], preferred_element_type=_f32) + blv_ref[...]
    z = mu + eps_ref[...] * jnp.exp(0.5 * lv)
    d = jnp.maximum(jnp.dot(z, d1w_ref[...], preferred_element_type=_f32)
                    + d1b_ref[...], 0.0)
    d = jnp.maximum(jnp.dot(d, d2w_ref[...], preferred_element_type=_f32)
                    + d2b_ref[...], 0.0)
    hd = jnp.dot(d, d3w_ref[...], preferred_element_type=_f32) + d3b_ref[...]
    z_ref[...] = z
    mu_ref[...] = mu
    lv_ref[...] = lv
    hd_ref[...] = hd


_tc_fin = pl.pallas_call(
    _tc_fin_body,
    out_shape=[
        jax.ShapeDtypeStruct((NG, LAT), _f32),
        jax.ShapeDtypeStruct((NG, LAT), _f32),
        jax.ShapeDtypeStruct((NG, LAT), _f32),
        jax.ShapeDtypeStruct((NG, 2 * H), _f32),
    ],
)


def _depack(o):
    o0 = o[:PKR].reshape(PKR * PK, 32)[:N]
    o1 = o[PKR:].reshape(PKR * PK, 32)[:N]
    return o0, o1


def kernel(x, edge_index, edge_attr, batch, W_in, b_in, Wg, bg, We,
           W_ih, W_hh, b_lstm, W_mu, b_mu, W_lv, b_lv,
           D1w, D1b, D2w, D2b, D3w, D3b):
    src = edge_index[0]
    dst = edge_index[1]
    pad = EP - E
    src_p = jnp.concatenate(
        [src, jnp.zeros((pad,), jnp.int32)]).reshape(NCHUNK, CH)
    dst_p = jnp.concatenate(
        [dst, jnp.full((pad,), N, jnp.int32)]).reshape(NCHUNK, CH)
    ea_t = jnp.concatenate(
        [edge_attr, jnp.zeros((pad, DE), _f32)], axis=0)
    sd_p = (src_p + dst_p * 16384).reshape(NCHUNK, 1, CH)
    zpk = jnp.zeros((PZR, HH), _f32)
    z128 = jnp.zeros((ZR, HH), _f32)
    ones_dis = jnp.ones((NPAD // CH, CH), _f32)

    dp0, dp1 = _depack(_sc_cs(src_p, dst_p, ea_t, ones_dis, zpk))
    dis_b, ht0, ht1 = _tc1(x, dp0, dp1, W_in, b_in.reshape(1, H))
    dis_pad = jnp.concatenate(
        [dis_b[:, 0], jnp.ones((NPAD - N,), _f32)]).reshape(NPAD // CH, CH)
    cs0, cs1 = _depack(_sc_cs(src_p, dst_p, ea_t, dis_pad, zpk))

    h = None
    for l in range(NL):
        table = jnp.concatenate([ht0, ht1], axis=0)
        pout = _sc_p(sd_p, table, z128)
        p0, p1 = pout[:N], pout[NPAD:NPAD + N]
        h, ht0, ht1 = _tc_layer(p0, p1, ht0, ht1, cs0, cs1, dis_b,
                                Wg[l], bg[l].reshape(1, H), We[l])

    eps = jax.random.normal(jax.random.key(42), (NG, LAT), dtype=_f32)
    z, mu, lv, hd = _tc_fin(
        h, batch.reshape(N, 1), W_ih, W_hh, b_lstm.reshape(1, 4 * H),
        W_mu, b_mu.reshape(1, LAT), W_lv, b_lv.reshape(1, LAT),
        D1w, D1b.reshape(1, H), D2w, D2b.reshape(1, H),
        D3w, D3b.reshape(1, 2 * H), eps)
    return (z, mu, lv, hd, h)
